# baseline (speedup 1.0000x reference)
import jax
import jax.numpy as jnp
from jax.experimental import pallas as pl
from jax.experimental.pallas import tpu as pltpu

B = 16384
DIM_OBS = 128
HID = 756
ACTION_DIM = 16
BM = 4096

_DNUMS = (((1,), (1,)), ((), ()))


def _mlp_block(state_ref, w1_ref, b1_ref, w2_ref, b2_ref, w3_ref, b3_ref,
               out_ref):

    x = state_ref[:]
    h = jax.lax.dot_general(x, w1_ref[:], _DNUMS,
                            preferred_element_type=jnp.float32)
    h = jnp.maximum(h + b1_ref[:], 0.0)
    h = jax.lax.dot_general(h, w2_ref[:], _DNUMS,
                            preferred_element_type=jnp.float32)
    h = jnp.maximum(h + b2_ref[:], 0.0)
    h = jax.lax.dot_general(h, w3_ref[:], _DNUMS,
                            preferred_element_type=jnp.float32)
    out_ref[:] = h + b3_ref[:]


@jax.jit
def kernel(state, W1, b1, W2, b2, W3, b3):
    grid = (B // BM,)
    fixed = lambda i: (0, 0)
    return pl.pallas_call(
        _mlp_block,
        grid=grid,
        in_specs=[
            pl.BlockSpec((BM, DIM_OBS), lambda i: (i, 0)),
            pl.BlockSpec((HID, DIM_OBS), fixed),
            pl.BlockSpec((1, HID), fixed),
            pl.BlockSpec((HID, HID), fixed),
            pl.BlockSpec((1, HID), fixed),
            pl.BlockSpec((ACTION_DIM, HID), fixed),
            pl.BlockSpec((1, ACTION_DIM), fixed),
        ],
        out_specs=pl.BlockSpec((BM, ACTION_DIM), lambda i: (i, 0)),
        out_shape=jax.ShapeDtypeStruct((B, ACTION_DIM), jnp.float32),
        compiler_params=pltpu.CompilerParams(
            dimension_semantics=("arbitrary",),
        ),
    )(state, W1, b1.reshape(1, HID), W2, b2.reshape(1, HID), W3,
      b3.reshape(1, ACTION_DIM))
